# Initial kernel scaffold; baseline (speedup 1.0000x reference)
import jax
import jax.numpy as jnp
from jax.experimental import pallas as pl


def kernel(x, w1_t, w2_t):
    raise NotImplementedError("write your pallas kernel here")


# speedup vs baseline: 1.3540x; 1.3540x over previous
import functools

import jax
import jax.numpy as jnp
from jax.experimental import pallas as pl
from jax.experimental.pallas import tpu as pltpu


def _se_kernel(x_ref, w1_ref, w2_ref, o_ref, *, inv_hw):
    x = x_ref[...]

    pooled = jnp.sum(x, axis=-1) * inv_hw
    h = jnp.maximum(
        jnp.dot(pooled, w1_ref[...], preferred_element_type=jnp.float32), 0.0)
    y = jax.nn.sigmoid(
        jnp.dot(h, w2_ref[...], preferred_element_type=jnp.float32))

    o_ref[...] = (x * y[:, :, None]).astype(o_ref.dtype)


def kernel(x, w1_t, w2_t):
    B, C, H, W = x.shape
    HW = H * W
    Cr = w1_t.shape[1]
    xr = x.reshape(B, C, HW)

    BB = 4
    while B % BB != 0:
        BB //= 2
    grid = (B // BB,)

    out = pl.pallas_call(
        functools.partial(_se_kernel, inv_hw=1.0 / HW),
        out_shape=jax.ShapeDtypeStruct((B, C, HW), x.dtype),
        grid_spec=pltpu.PrefetchScalarGridSpec(
            num_scalar_prefetch=0,
            grid=grid,
            in_specs=[
                pl.BlockSpec((BB, C, HW), lambda b: (b, 0, 0)),
                pl.BlockSpec((C, Cr), lambda b: (0, 0)),
                pl.BlockSpec((Cr, C), lambda b: (0, 0)),
            ],
            out_specs=pl.BlockSpec((BB, C, HW), lambda b: (b, 0, 0)),
        ),
        compiler_params=pltpu.CompilerParams(
            dimension_semantics=("parallel",),
            vmem_limit_bytes=64 * 1024 * 1024,
        ),
    )(xr, w1_t, w2_t)
    return out.reshape(B, C, H, W)


# speedup vs baseline: 1.4337x; 1.0589x over previous
import functools

import jax
import jax.numpy as jnp
from jax.experimental import pallas as pl
from jax.experimental.pallas import tpu as pltpu


def _se_kernel(x_ref, w1_ref, w2_ref, o_ref, *, inv_hw):
    x = x_ref[...]

    pooled = jnp.sum(x, axis=-1) * inv_hw
    h = jnp.maximum(
        jnp.dot(pooled, w1_ref[...], preferred_element_type=jnp.float32), 0.0)
    y = jax.nn.sigmoid(
        jnp.dot(h, w2_ref[...], preferred_element_type=jnp.float32))

    o_ref[...] = (x * y[:, :, None]).astype(o_ref.dtype)


def kernel(x, w1_t, w2_t):
    B, C, H, W = x.shape
    HW = H * W
    Cr = w1_t.shape[1]
    xr = x.reshape(B, C, HW)

    BB = 8
    while B % BB != 0:
        BB //= 2
    grid = (B // BB,)

    out = pl.pallas_call(
        functools.partial(_se_kernel, inv_hw=1.0 / HW),
        out_shape=jax.ShapeDtypeStruct((B, C, HW), x.dtype),
        grid_spec=pltpu.PrefetchScalarGridSpec(
            num_scalar_prefetch=0,
            grid=grid,
            in_specs=[
                pl.BlockSpec((BB, C, HW), lambda b: (b, 0, 0)),
                pl.BlockSpec((C, Cr), lambda b: (0, 0)),
                pl.BlockSpec((Cr, C), lambda b: (0, 0)),
            ],
            out_specs=pl.BlockSpec((BB, C, HW), lambda b: (b, 0, 0)),
        ),
        compiler_params=pltpu.CompilerParams(
            dimension_semantics=("parallel",),
            vmem_limit_bytes=64 * 1024 * 1024,
        ),
    )(xr, w1_t, w2_t)
    return out.reshape(B, C, H, W)
